# baseline (speedup 1.0000x reference)
import functools

import jax
import jax.numpy as jnp
from jax import lax
from jax.experimental import pallas as pl
from jax.experimental.pallas import tpu as pltpu
from jax.experimental.pallas import tpu_sc as plsc

N_NODES = 10000
SROWS = 10112
NC, NS, NW = 2, 16, 32
RPT = SROWS // NS
K = 512
KB = K // 128
E_TOT = 320000 + N_NODES
CPW = -(-E_TOT // (NW * K))
EP = NW * CPW * K
K3 = 384
KB3 = K3 // 128
CPW3 = -(-E_TOT // (NW * K3))
TOTCH = NW * CPW3
NTEST = 80000
DCH = -(-NTEST // (NW * K3))
EPD = NW * DCH * K3

def _mesh():
    return plsc.VectorSubcoreMesh(core_axis_name="c", subcore_axis_name="s",
                                  num_cores=NC, num_subcores=NS)


def _prep_body(x_ref, w_ref, ms_ref, md_ref, xp_ref, as_ref, ad_ref):
    xp = lax.dot_general(x_ref[...], w_ref[...], (((1,), (1,)), ((), ())),
                         preferred_element_type=jnp.float32)
    xp_ref[...] = xp
    as_ref[...] = jnp.dot(xp, ms_ref[...], preferred_element_type=jnp.float32)
    ad_ref[...] = jnp.dot(xp, md_ref[...], preferred_element_type=jnp.float32)


def _tc_prep(xin, W, Ms, Md):
    return pl.pallas_call(
        _prep_body,
        out_shape=(jax.ShapeDtypeStruct((SROWS, 64), jnp.float32),
                   jax.ShapeDtypeStruct((SROWS, 16), jnp.float32),
                   jax.ShapeDtypeStruct((SROWS, 16), jnp.float32)),
    )(xin, W, Ms, Md)


def _mid_body(op_ref, sp_ref, b_ref, w_ref, ms_ref, md_ref, r_ref,
              xp_ref, as_ref, ad_ref):
    raw = op_ref[0] + op_ref[1]
    s = sp_ref[0] + sp_ref[1]
    inv = 1.0 / (s + 1e-16)
    z = raw * jnp.dot(inv, r_ref[...], preferred_element_type=jnp.float32)
    z = z + b_ref[...]
    z = jnp.where(z > 0, z, jnp.exp(z) - 1.0)
    xp = lax.dot_general(z, w_ref[...], (((1,), (1,)), ((), ())),
                         preferred_element_type=jnp.float32)
    xp_ref[...] = xp
    as_ref[...] = jnp.dot(xp, ms_ref[...], preferred_element_type=jnp.float32)
    ad_ref[...] = jnp.dot(xp, md_ref[...], preferred_element_type=jnp.float32)


def _tc_mid(op, sp, b1, W2, Ms, Md, R8):
    return pl.pallas_call(
        _mid_body,
        out_shape=(jax.ShapeDtypeStruct((SROWS, 64), jnp.float32),
                   jax.ShapeDtypeStruct((SROWS, 16), jnp.float32),
                   jax.ShapeDtypeStruct((SROWS, 16), jnp.float32)),
    )(op, sp, b1, W2, Ms, Md, R8)


def _final_body(op_ref, sp_ref, b_ref, z_ref):
    raw = op_ref[0] + op_ref[1]
    s = sp_ref[0][:, 0:1] + sp_ref[1][:, 0:1]
    z_ref[...] = raw * (1.0 / (s + 1e-16)) + b_ref[...]


def _tc_final(op, sp, b2):
    return pl.pallas_call(
        _final_body,
        out_shape=jax.ShapeDtypeStruct((SROWS, 64), jnp.float32),
    )(op, sp, b2)


def _make_passA(H):
    @functools.partial(
        pl.kernel,
        mesh=_mesh(),
        compiler_params=pltpu.CompilerParams(needs_layout_passes=False, use_tc_tiling_on_sc=False),
        out_type=(jax.ShapeDtypeStruct((EP, 8), jnp.float32),
                  jax.ShapeDtypeStruct((NC, SROWS, 8), jnp.float32)),
        scratch_types=([pltpu.VMEM((KB, 128), jnp.int32)] * 6
                       + [pltpu.VMEM((K, 16), jnp.float32)] * 6
                       + [pltpu.VMEM((K, 8), jnp.float32)] * 3
                       + [pltpu.SemaphoreType.DMA] * 9
                       + [pltpu.VMEM_SHARED((SROWS, 8), jnp.float32)]),
    )
    def passA(src_ref, dst_ref, as_ref, ad_ref, z8_ref,
              ex_ref, sp_ref, *scr):
        i0 = scr[0:3]
        i1 = scr[3:6]
        g1 = scr[6:9]
        g2 = scr[9:12]
        exb = scr[12:15]
        gs = scr[15:18]
        ss = scr[18:21]
        es = scr[21:24]
        s_sh = scr[24]
        cid = lax.axis_index("c")
        sid = lax.axis_index("s")
        wid = sid * NC + cid
        r0 = sid * RPT
        pltpu.sync_copy(z8_ref, s_sh.at[pl.ds(r0, RPT)])
        for b in range(3):
            pltpu.sync_copy(z8_ref.at[pl.ds(0, K)], exb[b])
        plsc.subcore_barrier()

        lane = lax.broadcasted_iota(jnp.int32, (16,), 0)
        mask = lane < H
        col = lax.bitwise_and(lane, 7)

        def gather(c, b, wait):
            for j in range(KB):
                d1 = pltpu.make_async_copy(as_ref.at[i0[b].at[j]],
                                           g1[b].at[pl.ds(j * 128, 128)],
                                           gs[b])
                d2 = pltpu.make_async_copy(ad_ref.at[i1[b].at[j]],
                                           g2[b].at[pl.ds(j * 128, 128)],
                                           gs[b])
                if wait:
                    d1.wait()
                    d2.wait()
                else:
                    d1.start()
                    d2.start()

        def issue_gather(c, b):
            blk = (wid * CPW + c) * KB
            pltpu.sync_copy(src_ref.at[pl.ds(blk, KB)], i0[b])
            pltpu.sync_copy(dst_ref.at[pl.ds(blk, KB)], i1[b])
            gather(c, b, False)

        def out_ops(c, b, wait):
            base = (wid * CPW + c) * K
            d1 = pltpu.make_async_copy(exb[b], ex_ref.at[pl.ds(base, K)],
                                       es[b])
            if not wait:
                d1.start()
            else:
                d1.wait()
            for j in range(KB):
                d2 = pltpu.make_async_copy(exb[b].at[pl.ds(j * 128, 128)],
                                           s_sh.at[i1[b].at[j]], ss[b])
                if not wait:
                    d2.start(add=True)
                else:
                    d2.wait()

        def compute(b):
            def inner(g, carry):
                for e2 in range(16):
                    e = g * 16 + e2
                    v = g1[b][e, :] + g2[b][e, :]
                    v = jnp.where(v < 0, 0.2 * v, v)
                    ex = jnp.exp(v)
                    plsc.store_scatter(exb[b],
                                       [jnp.broadcast_to(e, (16,)), col],
                                       ex, mask=mask)
                return carry

            lax.fori_loop(0, K // 16, inner, 0)

        issue_gather(0, 0)

        def step(t, carry):
            for q in range(3):
                b, bn = q, (q + 1) % 3
                c = 3 * t + q
                if q == 2:
                    out_ops(c - 2, bn, True)

                    @pl.when(t < CPW // 3 - 1)
                    def _():
                        issue_gather(c + 1, bn)
                else:
                    @pl.when(t > 0)
                    def _():
                        out_ops(c - 2, bn, True)

                    issue_gather(c + 1, bn)
                gather(c, b, True)
                compute(b)
                out_ops(c, b, False)
            return carry

        lax.fori_loop(0, CPW // 3, step, 0)
        out_ops(CPW - 2, 1, True)
        out_ops(CPW - 1, 2, True)
        plsc.subcore_barrier()
        pltpu.sync_copy(s_sh.at[pl.ds(r0, RPT)],
                        sp_ref.at[cid, pl.ds(r0, RPT)])

    return passA


def _make_passB(H):
    @functools.partial(
        pl.kernel,
        mesh=_mesh(),
        compiler_params=pltpu.CompilerParams(needs_layout_passes=False, use_tc_tiling_on_sc=False),
        out_type=jax.ShapeDtypeStruct((NC, SROWS, 64), jnp.float32),
        scratch_types=([pltpu.VMEM((KB, 128), jnp.int32)] * 4
                       + [pltpu.VMEM((K, 64), jnp.float32)] * 2
                       + [pltpu.VMEM((K, 8), jnp.float32)] * 2
                       + [pltpu.SemaphoreType.DMA] * 6
                       + [pltpu.VMEM_SHARED((SROWS, 64), jnp.float32)]),
    )
    def passB(src_ref, dst_ref, xp_ref, ex_ref, z64_ref,
              op_ref, *scr):
        i0 = scr[0:2]
        i1 = scr[2:4]
        rows = scr[4:6]
        exb = scr[6:8]
        gs = scr[8:10]
        ss = scr[10:12]
        es = scr[12:14]
        out_sh = scr[14]
        cid = lax.axis_index("c")
        sid = lax.axis_index("s")
        wid = sid * NC + cid
        r0 = sid * RPT
        pltpu.sync_copy(z64_ref, out_sh.at[pl.ds(r0, RPT)])
        plsc.subcore_barrier()

        lane = lax.broadcasted_iota(jnp.int32, (16,), 0)
        cols = [2 * vv + (lane >= 8).astype(jnp.int32) for vv in range(4)]
        zcol = jnp.zeros((16,), jnp.int32)

        def gather(c, b, wait):
            base = (wid * CPW + c) * K
            dex = pltpu.make_async_copy(ex_ref.at[pl.ds(base, K)], exb[b],
                                        es[b])
            drows = [pltpu.make_async_copy(xp_ref.at[i0[b].at[j]],
                                           rows[b].at[pl.ds(j * 128, 128)],
                                           gs[b]) for j in range(KB)]
            if wait:
                dex.wait()
                for d in drows:
                    d.wait()
            else:
                dex.start()
                for d in drows:
                    d.start()

        def issue_gather(c, b):
            blk = (wid * CPW + c) * KB
            pltpu.sync_copy(src_ref.at[pl.ds(blk, KB)], i0[b])
            pltpu.sync_copy(dst_ref.at[pl.ds(blk, KB)], i1[b])
            gather(c, b, False)

        def scatter(b, wait):
            for j in range(KB):
                d = pltpu.make_async_copy(rows[b].at[pl.ds(j * 128, 128)],
                                          out_sh.at[i1[b].at[j]], ss[b])
                if wait:
                    d.wait()
                else:
                    d.start(add=True)

        def compute_scatter(b):
            def block(j, carry):
                def inner(g, carry2):
                    for e2 in range(16):
                        e = j * 128 + g * 16 + e2
                        es_ = jnp.broadcast_to(e, (16,))
                        if H == 8:
                            for vv in range(4):
                                exv = plsc.load_gather(exb[b],
                                                       [es_, cols[vv]])
                                rows[b][e, pl.ds(16 * vv, 16)] = (
                                    rows[b][e, pl.ds(16 * vv, 16)] * exv)
                        else:
                            exv = plsc.load_gather(exb[b], [es_, zcol])
                            for vv in range(4):
                                rows[b][e, pl.ds(16 * vv, 16)] = (
                                    rows[b][e, pl.ds(16 * vv, 16)] * exv)
                    return carry2

                lax.fori_loop(0, 8, inner, 0)
                pltpu.make_async_copy(rows[b].at[pl.ds(j * 128, 128)],
                                      out_sh.at[i1[b].at[j]],
                                      ss[b]).start(add=True)
                return carry

            lax.fori_loop(0, KB, block, 0)

        issue_gather(0, 0)

        def step(t, carry):
            for q in range(2):
                b, bn = q, 1 - q
                c = 2 * t + q
                gather(c, b, True)
                if q == 0:
                    @pl.when(t > 0)
                    def _():
                        scatter(bn, True)
                else:
                    scatter(bn, True)
                issue_gather(c + 1, bn)
                compute_scatter(b)
            return carry

        lax.fori_loop(0, (CPW - 1) // 2, step, 0)
        gather(CPW - 1, 0, True)
        scatter(1, True)
        compute_scatter(0)
        scatter(0, True)
        plsc.subcore_barrier()
        pltpu.sync_copy(out_sh.at[pl.ds(r0, RPT)],
                        op_ref.at[cid, pl.ds(r0, RPT)])

    return passB


def _make_fused(H):
    @functools.partial(
        pl.kernel,
        mesh=_mesh(),
        compiler_params=pltpu.CompilerParams(needs_layout_passes=False, use_tc_tiling_on_sc=False),
        out_type=(jax.ShapeDtypeStruct((NC, SROWS, 8), jnp.float32),
                  jax.ShapeDtypeStruct((NC, SROWS, 64), jnp.float32)),
        scratch_types=([pltpu.VMEM((8, 128), jnp.int32)] * 4
                       + [pltpu.VMEM((K3, 16), jnp.float32)] * 4
                       + [pltpu.VMEM((K3, 64), jnp.float32)] * 2
                       + [pltpu.VMEM((K3, 8), jnp.float32)] * 2
                       + [pltpu.SemaphoreType.DMA] * 4
                       + [pltpu.VMEM_SHARED((SROWS, 8), jnp.float32),
                          pltpu.VMEM_SHARED((SROWS, 64), jnp.float32)]),
    )
    def fused(src_ref, dst_ref, as_ref, ad_ref, xp_ref, z8_ref, z64_ref,
              sp_ref, op_ref, *scr):
        i0 = scr[0:2]
        i1 = scr[2:4]
        g1 = scr[4:6]
        g2 = scr[6:8]
        rows = scr[8:10]
        exb = scr[10:12]
        gs = scr[12:14]
        ss = scr[14:16]
        s_sh = scr[16]
        out_sh = scr[17]
        cid = lax.axis_index("c")
        sid = lax.axis_index("s")
        wid = sid * NC + cid
        r0 = sid * RPT
        pltpu.sync_copy(z8_ref, s_sh.at[pl.ds(r0, RPT)])
        pltpu.sync_copy(z64_ref, out_sh.at[pl.ds(r0, RPT)])
        for b in range(2):
            pltpu.sync_copy(z8_ref.at[pl.ds(0, K3)], exb[b])
        plsc.subcore_barrier()

        lane = lax.broadcasted_iota(jnp.int32, (16,), 0)
        mask = lane < H
        col = lax.bitwise_and(lane, 7)
        cols = [2 * vv + (lane >= 8).astype(jnp.int32) for vv in range(4)]
        zcol = jnp.zeros((16,), jnp.int32)

        def gather(c, b, wait):
            ds = []
            for j in range(KB3):
                ds.append(pltpu.make_async_copy(
                    as_ref.at[i0[b].at[j]],
                    g1[b].at[pl.ds(j * 128, 128)], gs[b]))
                ds.append(pltpu.make_async_copy(
                    ad_ref.at[i1[b].at[j]],
                    g2[b].at[pl.ds(j * 128, 128)], gs[b]))
                ds.append(pltpu.make_async_copy(
                    xp_ref.at[i0[b].at[j]],
                    rows[b].at[pl.ds(j * 128, 128)], gs[b]))
            for d in ds:
                d.wait() if wait else d.start()

        def issue_gather(c, b):
            blk = (wid * CPW3 + c) * 8
            pltpu.sync_copy(src_ref.at[pl.ds(blk, 8)], i0[b])
            pltpu.sync_copy(dst_ref.at[pl.ds(blk, 8)], i1[b])
            gather(c, b, False)

        def drain_scatter(b):
            for j in range(KB3):
                pltpu.make_async_copy(rows[b].at[pl.ds(j * 128, 128)],
                                      out_sh.at[i1[b].at[j]], ss[b]).wait()
                pltpu.make_async_copy(exb[b].at[pl.ds(j * 128, 128)],
                                      s_sh.at[i1[b].at[j]], ss[b]).wait()

        def compute_scatter(b):
            def block(j, carry):
                def inner(g, carry2):
                    for e2 in range(16):
                        e = j * 128 + g * 16 + e2
                        es_ = jnp.broadcast_to(e, (16,))
                        v = g1[b][e, :] + g2[b][e, :]
                        v = jnp.where(v < 0, 0.2 * v, v)
                        ex = jnp.exp(v)
                        plsc.store_scatter(exb[b], [es_, col], ex, mask=mask)
                        if H == 8:
                            for vv in range(4):
                                exv = plsc.load_gather(exb[b],
                                                       [es_, cols[vv]])
                                rows[b][e, pl.ds(16 * vv, 16)] = (
                                    rows[b][e, pl.ds(16 * vv, 16)] * exv)
                        else:
                            exv = plsc.load_gather(exb[b], [es_, zcol])
                            for vv in range(4):
                                rows[b][e, pl.ds(16 * vv, 16)] = (
                                    rows[b][e, pl.ds(16 * vv, 16)] * exv)
                    return carry2

                lax.fori_loop(0, 8, inner, 0)
                pltpu.make_async_copy(rows[b].at[pl.ds(j * 128, 128)],
                                      out_sh.at[i1[b].at[j]],
                                      ss[b]).start(add=True)
                pltpu.make_async_copy(exb[b].at[pl.ds(j * 128, 128)],
                                      s_sh.at[i1[b].at[j]],
                                      ss[b]).start(add=True)
                return carry

            lax.fori_loop(0, KB3, block, 0)

        issue_gather(0, 0)

        def step(t, carry):
            for q in range(2):
                b, bn = q, 1 - q
                c = 2 * t + q
                gather(c, b, True)
                if q == 0:
                    @pl.when(t > 0)
                    def _():
                        drain_scatter(bn)
                else:
                    drain_scatter(bn)
                issue_gather(c + 1, bn)
                compute_scatter(b)
            return carry

        lax.fori_loop(0, (CPW3 - 1) // 2, step, 0)
        gather(CPW3 - 1, 0, True)
        drain_scatter(1)
        compute_scatter(0)
        drain_scatter(0)
        plsc.subcore_barrier()
        pltpu.sync_copy(s_sh.at[pl.ds(r0, RPT)],
                        sp_ref.at[cid, pl.ds(r0, RPT)])
        pltpu.sync_copy(out_sh.at[pl.ds(r0, RPT)],
                        op_ref.at[cid, pl.ds(r0, RPT)])

    return fused


def _make_decode():
    @functools.partial(
        pl.kernel,
        mesh=_mesh(),
        compiler_params=pltpu.CompilerParams(needs_layout_passes=False, use_tc_tiling_on_sc=False),
        out_type=jax.ShapeDtypeStruct((EPD,), jnp.float32),
        scratch_types=([pltpu.VMEM((8, 128), jnp.int32)] * 4
                       + [pltpu.VMEM((K3, 64), jnp.float32)] * 4
                       + [pltpu.VMEM((K3,), jnp.float32)]
                       + [pltpu.SemaphoreType.DMA] * 2),
    )
    def decode(z_ref, t0_ref, t1_ref, lg_ref, *scr):
        i0 = scr[0:2]
        i1 = scr[2:4]
        r0b = scr[4:6]
        r1b = scr[6:8]
        lb = scr[8]
        gs = scr[9:11]
        cid = lax.axis_index("c")
        sid = lax.axis_index("s")
        wid = sid * NC + cid
        lane = lax.broadcasted_iota(jnp.int32, (16,), 0)

        def gather(c, b, wait):
            ds = []
            for j in range(KB3):
                ds.append(pltpu.make_async_copy(
                    z_ref.at[i0[b].at[j]],
                    r0b[b].at[pl.ds(j * 128, 128)], gs[b]))
                ds.append(pltpu.make_async_copy(
                    z_ref.at[i1[b].at[j]],
                    r1b[b].at[pl.ds(j * 128, 128)], gs[b]))
            for d in ds:
                d.wait() if wait else d.start()

        def issue_gather(c, b):
            blk = (wid * DCH + c) * 8
            pltpu.sync_copy(t0_ref.at[pl.ds(blk, 8)], i0[b])
            pltpu.sync_copy(t1_ref.at[pl.ds(blk, 8)], i1[b])
            gather(c, b, False)

        def compute_store(c, b):
            base = (wid * DCH + c) * K3
            def inner(g, carry):
                rowi = g * 16 + lane
                acc = jnp.zeros((16,), jnp.float32)
                for cc_ in range(64):
                    cc = jnp.full((16,), cc_, jnp.int32)
                    acc = acc + (plsc.load_gather(r0b[b], [rowi, cc]) *
                                 plsc.load_gather(r1b[b], [rowi, cc]))
                lb[pl.ds(g * 16, 16)] = acc
                return carry

            lax.fori_loop(0, K3 // 16, inner, 0)
            pltpu.sync_copy(lb, lg_ref.at[pl.ds(base, K3)])

        issue_gather(0, 0)

        def step(t, carry):
            for q in range(2):
                b, bn = q, 1 - q
                c = 2 * t + q
                gather(c, b, True)
                issue_gather(c + 1, bn)
                compute_store(c, b)
            return carry

        lax.fori_loop(0, (DCH - 1) // 2, step, 0)
        gather(DCH - 1, 0, True)
        compute_store(DCH - 1, 0)

    return decode


@functools.cache
def _sc_kernels():
    return (_make_fused(8), _make_fused(1), _make_decode())


def _att_mat8(a):
    t = a[0]
    m = (t[:, :, None] * jnp.eye(8, dtype=jnp.float32)[:, None, :]).reshape(64, 8)
    return jnp.pad(m, ((0, 0), (0, 8)))


def _att_mat1(a):
    return jnp.pad(a[0, 0][:, None], ((0, 0), (0, 15)))


def kernel(x, train_pos_edge_index, test_pos_edge_index, test_neg_edge_index,
           W1, a_src1, a_dst1, b1, W2, a_src2, a_dst2, b2):
    i32 = jnp.int32
    f32 = jnp.float32
    x_p = jnp.pad(x, ((0, SROWS - N_NODES), (0, 0)))
    loop = jnp.arange(N_NODES, dtype=i32)
    pad3 = jnp.full((TOTCH * K3 - E_TOT,), N_NODES, i32)
    src3 = jnp.concatenate([train_pos_edge_index[0], loop, pad3]
                           ).reshape(TOTCH, KB3, 128)
    dst3 = jnp.concatenate([train_pos_edge_index[1], loop, pad3]
                           ).reshape(TOTCH, KB3, 128)
    srcb = jnp.pad(src3, ((0, 0), (0, 8 - KB3), (0, 0))).reshape(TOTCH * 8,
                                                                 128)
    dstb = jnp.pad(dst3, ((0, 0), (0, 8 - KB3), (0, 0))).reshape(TOTCH * 8,
                                                                 128)
    padt = jnp.zeros((EPD - NTEST,), i32)
    t03 = jnp.concatenate([test_pos_edge_index[0], test_neg_edge_index[0],
                           padt]).reshape(NW * DCH, KB3, 128)
    t13 = jnp.concatenate([test_pos_edge_index[1], test_neg_edge_index[1],
                           padt]).reshape(NW * DCH, KB3, 128)
    t0 = jnp.pad(t03, ((0, 0), (0, 8 - KB3), (0, 0))).reshape(NW * DCH * 8,
                                                              128)
    t1 = jnp.pad(t13, ((0, 0), (0, 8 - KB3), (0, 0))).reshape(NW * DCH * 8,
                                                              128)

    Ms1, Md1 = _att_mat8(a_src1), _att_mat8(a_dst1)
    Ms2, Md2 = _att_mat1(a_src2), _att_mat1(a_dst2)
    R8 = jnp.repeat(jnp.eye(8, dtype=f32), 8, axis=1)
    z8 = jnp.zeros((RPT, 8), f32)
    z64 = jnp.zeros((RPT, 64), f32)
    b1r = b1.reshape(1, 64)
    b2r = b2.reshape(1, 64)

    fused8, fused1, decode = _sc_kernels()
    xp1, As1, Ad1 = _tc_prep(x_p, W1, Ms1, Md1)
    s1, o1 = fused8(srcb, dstb, As1, Ad1, xp1, z8, z64)
    xp2, As2, Ad2 = _tc_mid(o1, s1, b1r, W2, Ms2, Md2, R8)
    s2, o2 = fused1(srcb, dstb, As2, Ad2, xp2, z8, z64)
    z2 = _tc_final(o2, s2, b2r)
    logits = decode(z2, t0, t1)
    return logits[:NTEST]


# speedup vs baseline: 1.0566x; 1.0566x over previous
import functools

import jax
import jax.numpy as jnp
from jax import lax
from jax.experimental import pallas as pl
from jax.experimental.pallas import tpu as pltpu
from jax.experimental.pallas import tpu_sc as plsc

N_NODES = 10000
SROWS = 10112
NC, NS, NW = 2, 16, 32
RPT = SROWS // NS
K = 512
KB = K // 128
E_TOT = 320000 + N_NODES
CPW = -(-E_TOT // (NW * K))
EP = NW * CPW * K
K3 = 384
KB3 = K3 // 128
CPW3 = -(-E_TOT // (NW * K3))
TOTCH = NW * CPW3
NTEST = 80000
DCH = -(-NTEST // (NW * K))
EPD = NW * DCH * K

def _mesh():
    return plsc.VectorSubcoreMesh(core_axis_name="c", subcore_axis_name="s",
                                  num_cores=NC, num_subcores=NS)


def _prep_body(x_ref, w_ref, ms_ref, md_ref, xp_ref, as_ref, ad_ref):
    xp = lax.dot_general(x_ref[...], w_ref[...], (((1,), (1,)), ((), ())),
                         preferred_element_type=jnp.float32)
    xp_ref[...] = xp
    as_ref[...] = jnp.dot(xp, ms_ref[...], preferred_element_type=jnp.float32)
    ad_ref[...] = jnp.dot(xp, md_ref[...], preferred_element_type=jnp.float32)


def _tc_prep(xin, W, Ms, Md):
    return pl.pallas_call(
        _prep_body,
        out_shape=(jax.ShapeDtypeStruct((SROWS, 64), jnp.float32),
                   jax.ShapeDtypeStruct((SROWS, 16), jnp.float32),
                   jax.ShapeDtypeStruct((SROWS, 16), jnp.float32)),
    )(xin, W, Ms, Md)


def _mid_body(op_ref, sp_ref, b_ref, w_ref, ms_ref, md_ref, r_ref,
              xp_ref, as_ref, ad_ref):
    raw = op_ref[0] + op_ref[1]
    s = sp_ref[0] + sp_ref[1]
    inv = 1.0 / (s + 1e-16)
    z = raw * jnp.dot(inv, r_ref[...], preferred_element_type=jnp.float32)
    z = z + b_ref[...]
    z = jnp.where(z > 0, z, jnp.exp(z) - 1.0)
    xp = lax.dot_general(z, w_ref[...], (((1,), (1,)), ((), ())),
                         preferred_element_type=jnp.float32)
    xp_ref[...] = xp
    as_ref[...] = jnp.dot(xp, ms_ref[...], preferred_element_type=jnp.float32)
    ad_ref[...] = jnp.dot(xp, md_ref[...], preferred_element_type=jnp.float32)


def _tc_mid(op, sp, b1, W2, Ms, Md, R8):
    return pl.pallas_call(
        _mid_body,
        out_shape=(jax.ShapeDtypeStruct((SROWS, 64), jnp.float32),
                   jax.ShapeDtypeStruct((SROWS, 16), jnp.float32),
                   jax.ShapeDtypeStruct((SROWS, 16), jnp.float32)),
    )(op, sp, b1, W2, Ms, Md, R8)


def _final_body(op_ref, sp_ref, b_ref, z_ref):
    raw = op_ref[0] + op_ref[1]
    s = sp_ref[0][:, 0:1] + sp_ref[1][:, 0:1]
    z_ref[...] = raw * (1.0 / (s + 1e-16)) + b_ref[...]


def _tc_final(op, sp, b2):
    return pl.pallas_call(
        _final_body,
        out_shape=jax.ShapeDtypeStruct((SROWS, 64), jnp.float32),
    )(op, sp, b2)


def _make_passA(H):
    @functools.partial(
        pl.kernel,
        mesh=_mesh(),
        compiler_params=pltpu.CompilerParams(needs_layout_passes=False, use_tc_tiling_on_sc=False),
        out_type=(jax.ShapeDtypeStruct((EP, 8), jnp.float32),
                  jax.ShapeDtypeStruct((NC, SROWS, 8), jnp.float32)),
        scratch_types=([pltpu.VMEM((KB, 128), jnp.int32)] * 6
                       + [pltpu.VMEM((K, 16), jnp.float32)] * 6
                       + [pltpu.VMEM((K, 8), jnp.float32)] * 3
                       + [pltpu.SemaphoreType.DMA] * 9
                       + [pltpu.VMEM_SHARED((SROWS, 8), jnp.float32)]),
    )
    def passA(src_ref, dst_ref, as_ref, ad_ref, z8_ref,
              ex_ref, sp_ref, *scr):
        i0 = scr[0:3]
        i1 = scr[3:6]
        g1 = scr[6:9]
        g2 = scr[9:12]
        exb = scr[12:15]
        gs = scr[15:18]
        ss = scr[18:21]
        es = scr[21:24]
        s_sh = scr[24]
        cid = lax.axis_index("c")
        sid = lax.axis_index("s")
        wid = sid * NC + cid
        r0 = sid * RPT
        pltpu.sync_copy(z8_ref, s_sh.at[pl.ds(r0, RPT)])
        for b in range(3):
            pltpu.sync_copy(z8_ref.at[pl.ds(0, K)], exb[b])
        plsc.subcore_barrier()

        lane = lax.broadcasted_iota(jnp.int32, (16,), 0)
        mask = lane < H
        col = lax.bitwise_and(lane, 7)

        def gather(c, b, wait):
            for j in range(KB):
                d1 = pltpu.make_async_copy(as_ref.at[i0[b].at[j]],
                                           g1[b].at[pl.ds(j * 128, 128)],
                                           gs[b])
                d2 = pltpu.make_async_copy(ad_ref.at[i1[b].at[j]],
                                           g2[b].at[pl.ds(j * 128, 128)],
                                           gs[b])
                if wait:
                    d1.wait()
                    d2.wait()
                else:
                    d1.start()
                    d2.start()

        def issue_gather(c, b):
            blk = (wid * CPW + c) * KB
            pltpu.sync_copy(src_ref.at[pl.ds(blk, KB)], i0[b])
            pltpu.sync_copy(dst_ref.at[pl.ds(blk, KB)], i1[b])
            gather(c, b, False)

        def out_ops(c, b, wait):
            base = (wid * CPW + c) * K
            d1 = pltpu.make_async_copy(exb[b], ex_ref.at[pl.ds(base, K)],
                                       es[b])
            if not wait:
                d1.start()
            else:
                d1.wait()
            for j in range(KB):
                d2 = pltpu.make_async_copy(exb[b].at[pl.ds(j * 128, 128)],
                                           s_sh.at[i1[b].at[j]], ss[b])
                if not wait:
                    d2.start(add=True)
                else:
                    d2.wait()

        def compute(b):
            def inner(g, carry):
                for e2 in range(16):
                    e = g * 16 + e2
                    v = g1[b][e, :] + g2[b][e, :]
                    v = jnp.where(v < 0, 0.2 * v, v)
                    ex = jnp.exp(v)
                    plsc.store_scatter(exb[b],
                                       [jnp.broadcast_to(e, (16,)), col],
                                       ex, mask=mask)
                return carry

            lax.fori_loop(0, K // 16, inner, 0)

        issue_gather(0, 0)

        def step(t, carry):
            for q in range(3):
                b, bn = q, (q + 1) % 3
                c = 3 * t + q
                if q == 2:
                    out_ops(c - 2, bn, True)

                    @pl.when(t < CPW // 3 - 1)
                    def _():
                        issue_gather(c + 1, bn)
                else:
                    @pl.when(t > 0)
                    def _():
                        out_ops(c - 2, bn, True)

                    issue_gather(c + 1, bn)
                gather(c, b, True)
                compute(b)
                out_ops(c, b, False)
            return carry

        lax.fori_loop(0, CPW // 3, step, 0)
        out_ops(CPW - 2, 1, True)
        out_ops(CPW - 1, 2, True)
        plsc.subcore_barrier()
        pltpu.sync_copy(s_sh.at[pl.ds(r0, RPT)],
                        sp_ref.at[cid, pl.ds(r0, RPT)])

    return passA


def _make_passB(H):
    @functools.partial(
        pl.kernel,
        mesh=_mesh(),
        compiler_params=pltpu.CompilerParams(needs_layout_passes=False, use_tc_tiling_on_sc=False),
        out_type=jax.ShapeDtypeStruct((NC, SROWS, 64), jnp.float32),
        scratch_types=([pltpu.VMEM((KB, 128), jnp.int32)] * 4
                       + [pltpu.VMEM((K, 64), jnp.float32)] * 2
                       + [pltpu.VMEM((K, 8), jnp.float32)] * 2
                       + [pltpu.SemaphoreType.DMA] * 6
                       + [pltpu.VMEM_SHARED((SROWS, 64), jnp.float32)]),
    )
    def passB(src_ref, dst_ref, xp_ref, ex_ref, z64_ref,
              op_ref, *scr):
        i0 = scr[0:2]
        i1 = scr[2:4]
        rows = scr[4:6]
        exb = scr[6:8]
        gs = scr[8:10]
        ss = scr[10:12]
        es = scr[12:14]
        out_sh = scr[14]
        cid = lax.axis_index("c")
        sid = lax.axis_index("s")
        wid = sid * NC + cid
        r0 = sid * RPT
        pltpu.sync_copy(z64_ref, out_sh.at[pl.ds(r0, RPT)])
        plsc.subcore_barrier()

        lane = lax.broadcasted_iota(jnp.int32, (16,), 0)
        cols = [2 * vv + (lane >= 8).astype(jnp.int32) for vv in range(4)]
        zcol = jnp.zeros((16,), jnp.int32)

        def gather(c, b, wait):
            base = (wid * CPW + c) * K
            dex = pltpu.make_async_copy(ex_ref.at[pl.ds(base, K)], exb[b],
                                        es[b])
            drows = [pltpu.make_async_copy(xp_ref.at[i0[b].at[j]],
                                           rows[b].at[pl.ds(j * 128, 128)],
                                           gs[b]) for j in range(KB)]
            if wait:
                dex.wait()
                for d in drows:
                    d.wait()
            else:
                dex.start()
                for d in drows:
                    d.start()

        def issue_gather(c, b):
            blk = (wid * CPW + c) * KB
            pltpu.sync_copy(src_ref.at[pl.ds(blk, KB)], i0[b])
            pltpu.sync_copy(dst_ref.at[pl.ds(blk, KB)], i1[b])
            gather(c, b, False)

        def scatter(b, wait):
            for j in range(KB):
                d = pltpu.make_async_copy(rows[b].at[pl.ds(j * 128, 128)],
                                          out_sh.at[i1[b].at[j]], ss[b])
                if wait:
                    d.wait()
                else:
                    d.start(add=True)

        def compute_scatter(b):
            def block(j, carry):
                def inner(g, carry2):
                    for e2 in range(16):
                        e = j * 128 + g * 16 + e2
                        es_ = jnp.broadcast_to(e, (16,))
                        if H == 8:
                            for vv in range(4):
                                exv = plsc.load_gather(exb[b],
                                                       [es_, cols[vv]])
                                rows[b][e, pl.ds(16 * vv, 16)] = (
                                    rows[b][e, pl.ds(16 * vv, 16)] * exv)
                        else:
                            exv = plsc.load_gather(exb[b], [es_, zcol])
                            for vv in range(4):
                                rows[b][e, pl.ds(16 * vv, 16)] = (
                                    rows[b][e, pl.ds(16 * vv, 16)] * exv)
                    return carry2

                lax.fori_loop(0, 8, inner, 0)
                pltpu.make_async_copy(rows[b].at[pl.ds(j * 128, 128)],
                                      out_sh.at[i1[b].at[j]],
                                      ss[b]).start(add=True)
                return carry

            lax.fori_loop(0, KB, block, 0)

        issue_gather(0, 0)

        def step(t, carry):
            for q in range(2):
                b, bn = q, 1 - q
                c = 2 * t + q
                gather(c, b, True)
                if q == 0:
                    @pl.when(t > 0)
                    def _():
                        scatter(bn, True)
                else:
                    scatter(bn, True)
                issue_gather(c + 1, bn)
                compute_scatter(b)
            return carry

        lax.fori_loop(0, (CPW - 1) // 2, step, 0)
        gather(CPW - 1, 0, True)
        scatter(1, True)
        compute_scatter(0)
        scatter(0, True)
        plsc.subcore_barrier()
        pltpu.sync_copy(out_sh.at[pl.ds(r0, RPT)],
                        op_ref.at[cid, pl.ds(r0, RPT)])

    return passB


def _make_fused(H):
    @functools.partial(
        pl.kernel,
        mesh=_mesh(),
        compiler_params=pltpu.CompilerParams(needs_layout_passes=False, use_tc_tiling_on_sc=False),
        out_type=(jax.ShapeDtypeStruct((NC, SROWS, 8), jnp.float32),
                  jax.ShapeDtypeStruct((NC, SROWS, 64), jnp.float32)),
        scratch_types=([pltpu.VMEM((8, 128), jnp.int32)] * 4
                       + [pltpu.VMEM((K3, 16), jnp.float32)] * 4
                       + [pltpu.VMEM((K3, 64), jnp.float32)] * 2
                       + [pltpu.VMEM((K3, 8), jnp.float32)] * 2
                       + [pltpu.SemaphoreType.DMA] * 4
                       + [pltpu.VMEM_SHARED((SROWS, 8), jnp.float32),
                          pltpu.VMEM_SHARED((SROWS, 64), jnp.float32)]),
    )
    def fused(src_ref, dst_ref, as_ref, ad_ref, xp_ref, z8_ref, z64_ref,
              sp_ref, op_ref, *scr):
        i0 = scr[0:2]
        i1 = scr[2:4]
        g1 = scr[4:6]
        g2 = scr[6:8]
        rows = scr[8:10]
        exb = scr[10:12]
        gs = scr[12:14]
        ss = scr[14:16]
        s_sh = scr[16]
        out_sh = scr[17]
        cid = lax.axis_index("c")
        sid = lax.axis_index("s")
        wid = sid * NC + cid
        r0 = sid * RPT
        pltpu.sync_copy(z8_ref, s_sh.at[pl.ds(r0, RPT)])
        pltpu.sync_copy(z64_ref, out_sh.at[pl.ds(r0, RPT)])
        for b in range(2):
            pltpu.sync_copy(z8_ref.at[pl.ds(0, K3)], exb[b])
        plsc.subcore_barrier()

        lane = lax.broadcasted_iota(jnp.int32, (16,), 0)
        mask = lane < H
        col = lax.bitwise_and(lane, 7)
        cols = [2 * vv + (lane >= 8).astype(jnp.int32) for vv in range(4)]
        zcol = jnp.zeros((16,), jnp.int32)

        def gather(c, b, wait):
            ds = []
            for j in range(KB3):
                ds.append(pltpu.make_async_copy(
                    as_ref.at[i0[b].at[j]],
                    g1[b].at[pl.ds(j * 128, 128)], gs[b]))
                ds.append(pltpu.make_async_copy(
                    ad_ref.at[i1[b].at[j]],
                    g2[b].at[pl.ds(j * 128, 128)], gs[b]))
                ds.append(pltpu.make_async_copy(
                    xp_ref.at[i0[b].at[j]],
                    rows[b].at[pl.ds(j * 128, 128)], gs[b]))
            for d in ds:
                d.wait() if wait else d.start()

        def issue_gather(c, b):
            blk = (wid * CPW3 + c) * 8
            pltpu.sync_copy(src_ref.at[pl.ds(blk, 8)], i0[b])
            pltpu.sync_copy(dst_ref.at[pl.ds(blk, 8)], i1[b])
            gather(c, b, False)

        def drain_scatter(b):
            for j in range(KB3):
                pltpu.make_async_copy(rows[b].at[pl.ds(j * 128, 128)],
                                      out_sh.at[i1[b].at[j]], ss[b]).wait()
                pltpu.make_async_copy(exb[b].at[pl.ds(j * 128, 128)],
                                      s_sh.at[i1[b].at[j]], ss[b]).wait()

        def compute_scatter(b):
            def block(j, carry):
                def inner(g, carry2):
                    for e2 in range(16):
                        e = j * 128 + g * 16 + e2
                        es_ = jnp.broadcast_to(e, (16,))
                        v = g1[b][e, :] + g2[b][e, :]
                        v = jnp.where(v < 0, 0.2 * v, v)
                        ex = jnp.exp(v)
                        plsc.store_scatter(exb[b], [es_, col], ex, mask=mask)
                        if H == 8:
                            for vv in range(4):
                                exv = plsc.load_gather(exb[b],
                                                       [es_, cols[vv]])
                                rows[b][e, pl.ds(16 * vv, 16)] = (
                                    rows[b][e, pl.ds(16 * vv, 16)] * exv)
                        else:
                            exv = plsc.load_gather(exb[b], [es_, zcol])
                            for vv in range(4):
                                rows[b][e, pl.ds(16 * vv, 16)] = (
                                    rows[b][e, pl.ds(16 * vv, 16)] * exv)
                    return carry2

                lax.fori_loop(0, 8, inner, 0)
                pltpu.make_async_copy(rows[b].at[pl.ds(j * 128, 128)],
                                      out_sh.at[i1[b].at[j]],
                                      ss[b]).start(add=True)
                pltpu.make_async_copy(exb[b].at[pl.ds(j * 128, 128)],
                                      s_sh.at[i1[b].at[j]],
                                      ss[b]).start(add=True)
                return carry

            lax.fori_loop(0, KB3, block, 0)

        issue_gather(0, 0)

        def step(t, carry):
            for q in range(2):
                b, bn = q, 1 - q
                c = 2 * t + q
                gather(c, b, True)
                if q == 0:
                    @pl.when(t > 0)
                    def _():
                        drain_scatter(bn)
                else:
                    drain_scatter(bn)
                issue_gather(c + 1, bn)
                compute_scatter(b)
            return carry

        lax.fori_loop(0, (CPW3 - 1) // 2, step, 0)
        gather(CPW3 - 1, 0, True)
        drain_scatter(1)
        compute_scatter(0)
        drain_scatter(0)
        plsc.subcore_barrier()
        pltpu.sync_copy(s_sh.at[pl.ds(r0, RPT)],
                        sp_ref.at[cid, pl.ds(r0, RPT)])
        pltpu.sync_copy(out_sh.at[pl.ds(r0, RPT)],
                        op_ref.at[cid, pl.ds(r0, RPT)])

    return fused


def _make_decode():
    @functools.partial(
        pl.kernel,
        mesh=_mesh(),
        compiler_params=pltpu.CompilerParams(needs_layout_passes=False, use_tc_tiling_on_sc=False),
        out_type=jax.ShapeDtypeStruct((EPD,), jnp.float32),
        scratch_types=([pltpu.VMEM((KB, 128), jnp.int32)] * 2
                       + [pltpu.VMEM((K, 64), jnp.float32)] * 2
                       + [pltpu.VMEM((K,), jnp.float32)]
                       + [pltpu.SemaphoreType.DMA] * 8),
    )
    def decode(z_ref, t0_ref, t1_ref, lg_ref, i0, i1, r0b, r1b, lb, *sems):
        cid = lax.axis_index("c")
        sid = lax.axis_index("s")
        wid = sid * NC + cid
        lane = lax.broadcasted_iota(jnp.int32, (16,), 0)

        def chunk(ci, _):
            blk = (wid * DCH + ci) * KB
            base = (wid * DCH + ci) * K
            pltpu.sync_copy(t0_ref.at[pl.ds(blk, KB)], i0)
            pltpu.sync_copy(t1_ref.at[pl.ds(blk, KB)], i1)
            ds = []
            for j in range(KB):
                d0 = pltpu.make_async_copy(z_ref.at[i0.at[j]],
                                           r0b.at[pl.ds(j * 128, 128)],
                                           sems[2 * j])
                d1 = pltpu.make_async_copy(z_ref.at[i1.at[j]],
                                           r1b.at[pl.ds(j * 128, 128)],
                                           sems[2 * j + 1])
                d0.start()
                d1.start()
                ds.append((d0, d1))
            for j in range(KB):
                ds[j][0].wait()
                ds[j][1].wait()

                def inner(g, carry):
                    rowi = j * 128 + g * 16 + lane
                    acc = jnp.zeros((16,), jnp.float32)
                    for c in range(64):
                        cc = jnp.full((16,), c, jnp.int32)
                        acc = acc + (plsc.load_gather(r0b, [rowi, cc]) *
                                     plsc.load_gather(r1b, [rowi, cc]))
                    lb[pl.ds(j * 128 + g * 16, 16)] = acc
                    return carry

                lax.fori_loop(0, 8, inner, 0)
            pltpu.sync_copy(lb, lg_ref.at[pl.ds(base, K)])
            return _

        lax.fori_loop(0, DCH, chunk, 0)

    return decode


@functools.cache
def _sc_kernels():
    return (_make_fused(8), _make_fused(1), _make_decode())


def _att_mat8(a):
    t = a[0]
    m = (t[:, :, None] * jnp.eye(8, dtype=jnp.float32)[:, None, :]).reshape(64, 8)
    return jnp.pad(m, ((0, 0), (0, 8)))


def _att_mat1(a):
    return jnp.pad(a[0, 0][:, None], ((0, 0), (0, 15)))


def kernel(x, train_pos_edge_index, test_pos_edge_index, test_neg_edge_index,
           W1, a_src1, a_dst1, b1, W2, a_src2, a_dst2, b2):
    i32 = jnp.int32
    f32 = jnp.float32
    x_p = jnp.pad(x, ((0, SROWS - N_NODES), (0, 0)))
    loop = jnp.arange(N_NODES, dtype=i32)
    pad3 = jnp.full((TOTCH * K3 - E_TOT,), N_NODES, i32)
    src3 = jnp.concatenate([train_pos_edge_index[0], loop, pad3]
                           ).reshape(TOTCH, KB3, 128)
    dst3 = jnp.concatenate([train_pos_edge_index[1], loop, pad3]
                           ).reshape(TOTCH, KB3, 128)
    srcb = jnp.pad(src3, ((0, 0), (0, 8 - KB3), (0, 0))).reshape(TOTCH * 8,
                                                                 128)
    dstb = jnp.pad(dst3, ((0, 0), (0, 8 - KB3), (0, 0))).reshape(TOTCH * 8,
                                                                 128)
    padt = jnp.zeros((EPD - NTEST,), i32)
    t0 = jnp.concatenate([test_pos_edge_index[0], test_neg_edge_index[0], padt]
                         ).reshape(EPD // 128, 128)
    t1 = jnp.concatenate([test_pos_edge_index[1], test_neg_edge_index[1], padt]
                         ).reshape(EPD // 128, 128)

    Ms1, Md1 = _att_mat8(a_src1), _att_mat8(a_dst1)
    Ms2, Md2 = _att_mat1(a_src2), _att_mat1(a_dst2)
    R8 = jnp.repeat(jnp.eye(8, dtype=f32), 8, axis=1)
    z8 = jnp.zeros((RPT, 8), f32)
    z64 = jnp.zeros((RPT, 64), f32)
    b1r = b1.reshape(1, 64)
    b2r = b2.reshape(1, 64)

    fused8, fused1, decode = _sc_kernels()
    xp1, As1, Ad1 = _tc_prep(x_p, W1, Ms1, Md1)
    s1, o1 = fused8(srcb, dstb, As1, Ad1, xp1, z8, z64)
    xp2, As2, Ad2 = _tc_mid(o1, s1, b1r, W2, Ms2, Md2, R8)
    s2, o2 = fused1(srcb, dstb, As2, Ad2, xp2, z8, z64)
    z2 = _tc_final(o2, s2, b2r)
    logits = decode(z2, t0, t1)
    return logits[:NTEST]
